# baseline (speedup 1.0000x reference)
import functools

import jax
import jax.numpy as jnp
from jax import lax
from jax.experimental import pallas as pl
from jax.experimental.pallas import tpu as pltpu
from jax.experimental.pallas import tpu_sc as plsc

B, L, N, H = 64, 128, 32, 128
BL = B * L
NP1 = N + 1
NUM_ROWS = BL * NP1
V = 100001
PQ = 16
NW = 32
PER_W = NUM_ROWS // NW
NODES_W = BL // NW
NB = 8
BROWS = NB * NP1
NBATCH = NODES_W // NB


def _mk_mesh():
    return plsc.VectorSubcoreMesh(
        core_axis_name="c", subcore_axis_name="s", num_cores=2, num_subcores=16
    )


def _wid():
    return lax.axis_index("s") * 2 + lax.axis_index("c")


_GDN = lax.GatherDimensionNumbers(
    offset_dims=(), collapsed_slice_dims=(0,), start_index_map=(0,))


def _lane_shuffle(v, perm):
    return lax.gather(v, perm[:, None], _GDN, (1,),
                      mode=lax.GatherScatterMode.PROMISE_IN_BOUNDS)


def _bcast_reduce(v, op):
    for shift in (1, 2, 4, 8):
        perm = lax.iota(jnp.int32, 16) ^ shift
        v = op(v, _lane_shuffle(v, perm))
    return v


def _pq_body(emb_ref, aa_ref, abcol_ref, out_ref):
    out_ref[...] = lax.dot_general(
        aa_ref[...], emb_ref[...], (((0,), (1,)), ((), ())),
        preferred_element_type=jnp.float32) + abcol_ref[...]


def _tc_pq(emb, aa, abcol):
    R = 4096
    grid = (pl.cdiv(V, R),)
    return pl.pallas_call(
        _pq_body,
        grid=grid,
        in_specs=[
            pl.BlockSpec((R, H), lambda i: (i, 0)),
            pl.BlockSpec((H, PQ), lambda i: (0, 0)),
            pl.BlockSpec((PQ, 1), lambda i: (0, 0)),
        ],
        out_specs=pl.BlockSpec((PQ, R), lambda i: (0, i)),
        out_shape=jax.ShapeDtypeStruct((PQ, V), jnp.float32),
    )(emb, aa, abcol)


SLOT = 64
IB = NB * SLOT
PER_W_I = NBATCH * IB
MW = 48


def _sc_attention(emb, pq1d, idx2, idxr, mask3):
    @functools.partial(
        pl.kernel,
        out_type=jax.ShapeDtypeStruct((BL, H), jnp.float32),
        mesh=_mk_mesh(),
        scratch_types=[
            pltpu.VMEM((NODES_W, MW), jnp.float32),
            pltpu.VMEM((IB,), jnp.int32),
            pltpu.VMEM((IB,), jnp.int32),
            pltpu.VMEM((BROWS,), jnp.int32),
            pltpu.VMEM((BROWS,), jnp.int32),
            pltpu.VMEM((IB,), jnp.float32),
            pltpu.VMEM((IB,), jnp.float32),
            pltpu.VMEM((2, BROWS, H), jnp.float32),
            pltpu.VMEM((NB, H), jnp.float32),
            pltpu.VMEM((MW,), jnp.float32),
            pltpu.SemaphoreType.DMA,
            pltpu.SemaphoreType.DMA,
            pltpu.SemaphoreType.DMA,
            pltpu.SemaphoreType.DMA,
        ],
    )
    def k(emb_hbm, pq_hbm, idx2_hbm, idxr_hbm, mask_hbm, out_hbm,
          mask_v, idx_v0, idx_v1, idxr_v0, idxr_v1, p_v0, p_v1, rows_v, out_v,
          ew_v, gsem0, gsem1, psem0, psem1):
        wid = _wid()
        ibase = wid * PER_W_I
        rbase = wid * PER_W
        nbase = wid * NODES_W
        idxs = (idx_v0, idx_v1)
        idxrs = (idxr_v0, idxr_v1)
        pvs = (p_v0, p_v1)
        gsems = (gsem0, gsem1)
        psems = (psem0, psem1)

        pltpu.sync_copy(mask_hbm.at[pl.ds(nbase, NODES_W)], mask_v)

        def issue(bi, s):
            pltpu.sync_copy(idx2_hbm.at[pl.ds(ibase + bi * IB, IB)], idxs[s])
            pltpu.sync_copy(
                idxr_hbm.at[pl.ds(rbase + bi * BROWS, BROWS)], idxrs[s])
            pltpu.async_copy(emb_hbm.at[idxrs[s]], rows_v.at[s], gsems[s])
            pltpu.async_copy(pq_hbm.at[idxs[s]], pvs[s], psems[s])

        def compute(bi, s):
            pltpu.make_async_copy(
                emb_hbm.at[idxrs[s]], rows_v.at[s], gsems[s]).wait()
            pltpu.make_async_copy(
                pq_hbm.at[idxs[s]], pvs[s], psems[s]).wait()

            def node(nl, carry):
                nrow = bi * NB + nl
                sb = pl.multiple_of(nl * SLOT, 16)
                qv = pvs[s][pl.ds(sb, 16)]
                q0 = jnp.full((16,), qv[0], jnp.float32)
                sv = []
                for t in range(3):
                    off = pl.multiple_of(nl * SLOT + 16 * (t + 1), 16)
                    s_t = pvs[s][pl.ds(off, 16)] + q0
                    s_t = jnp.where(s_t >= 0, s_t, 0.2 * s_t)
                    s_t = s_t + mask_v[nrow, pl.ds(16 * t, 16)] * (-1e9)
                    sv.append(s_t)
                mxv = _bcast_reduce(
                    jnp.maximum(jnp.maximum(sv[0], sv[1]), sv[2]),
                    jnp.maximum)
                e0 = jnp.exp(sv[0] - mxv)
                e1 = jnp.exp(sv[1] - mxv)
                e2 = jnp.exp(sv[2] - mxv)
                rinv = 1.0 / _bcast_reduce(e0 + e1 + e2, jnp.add)
                ew_v[pl.ds(0, 16)] = e0 * rinv
                ew_v[pl.ds(16, 16)] = e1 * rinv
                ew_v[pl.ds(32, 16)] = e2 * rinv
                wvec = (ew_v[pl.ds(0, 16)], ew_v[pl.ds(16, 16)],
                        ew_v[pl.ds(32, 16)])

                acc = tuple(
                    jnp.zeros((16,), jnp.float32) for _ in range(H // 16))
                pb = nl * NP1
                for j in range(NP1):
                    wj = wvec[j // 16][j % 16]
                    wv = jnp.full((16,), wj, jnp.float32)
                    row = pb + j
                    acc = tuple(
                        acc[t] + wv * rows_v[s, row, pl.ds(t * 16, 16)]
                        for t in range(H // 16)
                    )
                for t in range(H // 16):
                    out_v[nl, pl.ds(t * 16, 16)] = acc[t]
                return carry

            lax.fori_loop(0, NB, node, 0)

            pltpu.sync_copy(out_v, out_hbm.at[pl.ds(nbase + bi * NB, NB)])

        issue(0, 0)
        issue(1, 1)

        def body(b2, carry):
            for s in range(2):
                bi = b2 * 2 + s
                compute(bi, s)
                issue(bi + 2, s)
            return carry

        lax.fori_loop(0, NBATCH // 2 - 1, body, 0)
        compute(NBATCH - 2, 0)
        compute(NBATCH - 1, 1)

    return k(emb, pq1d, idx2, idxr, mask3)


def _mlp_body(node_ref, ls_ref, fc1e_ref, fc1s_ref, fc1b_ref, fc2_ref,
              fc2b_ref, out_ref):
    h = (jnp.dot(node_ref[...], fc1e_ref[...],
                 preferred_element_type=jnp.float32)
         + jnp.dot(ls_ref[...], fc1s_ref[...],
                   preferred_element_type=jnp.float32)
         + fc1b_ref[...])
    h = jnp.where(h >= 0, h, 0.2 * h)
    o = (jnp.dot(h, fc2_ref[...], preferred_element_type=jnp.float32)
         + fc2b_ref[...])
    out_ref[...] = jnp.where(o >= 0, o, 0.2 * o)


def _tc_mlp(node, ls, fc1e, fc1s, fc1b, fc2w, fc2b):
    R = 512
    grid = (BL // R,)
    full = lambda i: (0, 0)
    return pl.pallas_call(
        _mlp_body,
        grid=grid,
        in_specs=[
            pl.BlockSpec((R, H), lambda i: (i, 0)),
            pl.BlockSpec((R, 16), lambda i: (i, 0)),
            pl.BlockSpec((H, H), full),
            pl.BlockSpec((16, H), full),
            pl.BlockSpec((1, H), full),
            pl.BlockSpec((H, H), full),
            pl.BlockSpec((1, H), full),
        ],
        out_specs=pl.BlockSpec((R, H), lambda i: (i, 0)),
        out_shape=jax.ShapeDtypeStruct((BL, H), jnp.float32),
    )(node, ls, fc1e, fc1s, fc1b, fc2w, fc2b)


def kernel(subgraph, neighs, mask, local_stats, global_stats, extra, emb,
           a_w, a_b, fc1_w, fc1_b, fc2_w, fc2_b):
    idx = jnp.concatenate(
        [subgraph.reshape(BL, 1), neighs.reshape(BL, N)], axis=1
    ).reshape(NUM_ROWS).astype(jnp.int32)
    aa = jnp.concatenate(
        [a_w[H:], a_w[:H], jnp.zeros((H, PQ - 2), jnp.float32)], axis=1)
    abcol = jnp.zeros((PQ, 1), jnp.float32).at[1, 0].set(a_b[0])
    pqT = _tc_pq(emb, aa, abcol)
    pq1d = pqT[0:2].reshape(2 * V)
    idx2 = jnp.concatenate(
        [subgraph.reshape(BL, 1).astype(jnp.int32) + V,
         jnp.zeros((BL, 15), jnp.int32),
         idx.reshape(BL, NP1),
         jnp.zeros((BL, SLOT - 16 - NP1), jnp.int32)],
        axis=1).reshape(BL * SLOT)
    mask3 = jnp.concatenate(
        [jnp.zeros((BL, 1), jnp.float32), mask.reshape(BL, N),
         jnp.ones((BL, MW - NP1), jnp.float32)], axis=1)
    node = _sc_attention(emb, pq1d, idx2, idx, mask3)
    ls = jnp.concatenate(
        [local_stats.reshape(BL, 8),
         jnp.broadcast_to(global_stats, (B, L, 1)).reshape(BL, 1),
         jnp.zeros((BL, 7), jnp.float32)], axis=1)
    fc1e = fc1_w[:H]
    fc1s = jnp.concatenate([fc1_w[H:], jnp.zeros((7, H), jnp.float32)], axis=0)
    out = _tc_mlp(node, ls, fc1e, fc1s, fc1_b.reshape(1, H), fc2_w,
                  fc2_b.reshape(1, H))
    return out.reshape(B, L, H)


# speedup vs baseline: 5.9838x; 5.9838x over previous
import functools

import jax
import jax.numpy as jnp
from jax import lax
from jax.experimental import pallas as pl
from jax.experimental.pallas import tpu as pltpu
from jax.experimental.pallas import tpu_sc as plsc

B, L, N, H = 64, 128, 32, 128
BL = B * L
NP1 = N + 1
NUM_ROWS = BL * NP1
V = 100001
PQ = 16
NW = 32
PER_W = NUM_ROWS // NW
NODES_W = BL // NW
NB = 8
BROWS = NB * NP1
NBATCH = NODES_W // NB


def _mk_mesh():
    return plsc.VectorSubcoreMesh(
        core_axis_name="c", subcore_axis_name="s", num_cores=2, num_subcores=16
    )


def _wid():
    return lax.axis_index("s") * 2 + lax.axis_index("c")


_GDN = lax.GatherDimensionNumbers(
    offset_dims=(), collapsed_slice_dims=(0,), start_index_map=(0,))


def _lane_shuffle(v, perm):
    return lax.gather(v, perm[:, None], _GDN, (1,),
                      mode=lax.GatherScatterMode.PROMISE_IN_BOUNDS)


def _bcast_reduce(v, op):
    for shift in (1, 2, 4, 8):
        perm = lax.iota(jnp.int32, 16) ^ shift
        v = op(v, _lane_shuffle(v, perm))
    return v


def _pq_body(emb_ref, aa_ref, abcol_ref, out_ref):
    out_ref[...] = lax.dot_general(
        aa_ref[...], emb_ref[...], (((0,), (1,)), ((), ())),
        preferred_element_type=jnp.float32) + abcol_ref[...]


def _tc_pq(emb, aa, abcol):
    R = 4096
    grid = (pl.cdiv(V, R),)
    return pl.pallas_call(
        _pq_body,
        grid=grid,
        in_specs=[
            pl.BlockSpec((R, H), lambda i: (i, 0)),
            pl.BlockSpec((H, PQ), lambda i: (0, 0)),
            pl.BlockSpec((PQ, 1), lambda i: (0, 0)),
        ],
        out_specs=pl.BlockSpec((PQ, R), lambda i: (0, i)),
        out_shape=jax.ShapeDtypeStruct((PQ, V), jnp.float32),
    )(emb, aa, abcol)


SLOT = 64
IB = NB * SLOT
PER_W_I = NBATCH * IB
MW = 48


def _sc_attention(emb, pq1d, idx2, idxr, mask3):
    @functools.partial(
        pl.kernel,
        out_type=jax.ShapeDtypeStruct((BL, H), jnp.float32),
        mesh=_mk_mesh(),
        scratch_types=[
            pltpu.VMEM((NODES_W, MW), jnp.float32),
            pltpu.VMEM((IB,), jnp.int32),
            pltpu.VMEM((IB,), jnp.int32),
            pltpu.VMEM((BROWS,), jnp.int32),
            pltpu.VMEM((BROWS,), jnp.int32),
            pltpu.VMEM((IB,), jnp.float32),
            pltpu.VMEM((IB,), jnp.float32),
            pltpu.VMEM((2, BROWS, H), jnp.float32),
            pltpu.VMEM((NB, H), jnp.float32),
            pltpu.VMEM((MW,), jnp.float32),
            pltpu.SemaphoreType.DMA,
            pltpu.SemaphoreType.DMA,
            pltpu.SemaphoreType.DMA,
            pltpu.SemaphoreType.DMA,
        ],
    )
    def k(emb_hbm, pq_hbm, idx2_hbm, idxr_hbm, mask_hbm, out_hbm,
          mask_v, idx_v0, idx_v1, idxr_v0, idxr_v1, p_v0, p_v1, rows_v, out_v,
          ew_v, gsem0, gsem1, psem0, psem1):
        wid = _wid()
        ibase = wid * PER_W_I
        rbase = wid * PER_W
        nbase = wid * NODES_W
        idxs = (idx_v0, idx_v1)
        idxrs = (idxr_v0, idxr_v1)
        pvs = (p_v0, p_v1)
        gsems = (gsem0, gsem1)
        psems = (psem0, psem1)

        pltpu.sync_copy(mask_hbm.at[pl.ds(nbase, NODES_W)], mask_v)

        def issue(bi, s):
            pltpu.sync_copy(idx2_hbm.at[pl.ds(ibase + bi * IB, IB)], idxs[s])
            pltpu.sync_copy(
                idxr_hbm.at[pl.ds(rbase + bi * BROWS, BROWS)], idxrs[s])
            pltpu.async_copy(emb_hbm.at[idxrs[s]], rows_v.at[s], gsems[s])
            pltpu.async_copy(pq_hbm.at[idxs[s]], pvs[s], psems[s])

        def compute(bi, s):
            pltpu.make_async_copy(
                emb_hbm.at[idxrs[s]], rows_v.at[s], gsems[s]).wait()
            pltpu.make_async_copy(
                pq_hbm.at[idxs[s]], pvs[s], psems[s]).wait()

            def node(nl, carry):
                nrow = bi * NB + nl
                sb = pl.multiple_of(nl * SLOT, 16)
                qv = pvs[s][pl.ds(sb, 16)]
                q0 = jnp.full((16,), qv[0], jnp.float32)
                sv = []
                for t in range(3):
                    off = pl.multiple_of(nl * SLOT + 16 * (t + 1), 16)
                    s_t = pvs[s][pl.ds(off, 16)] + q0
                    s_t = jnp.where(s_t >= 0, s_t, 0.2 * s_t)
                    s_t = s_t + mask_v[nrow, pl.ds(16 * t, 16)] * (-1e9)
                    sv.append(s_t)
                mxv = _bcast_reduce(
                    jnp.maximum(jnp.maximum(sv[0], sv[1]), sv[2]),
                    jnp.maximum)
                e0 = jnp.exp(sv[0] - mxv)
                e1 = jnp.exp(sv[1] - mxv)
                e2 = jnp.exp(sv[2] - mxv)
                rinv = 1.0 / _bcast_reduce(e0 + e1 + e2, jnp.add)
                ew_v[pl.ds(0, 16)] = e0 * rinv
                ew_v[pl.ds(16, 16)] = e1 * rinv
                ew_v[pl.ds(32, 16)] = e2 * rinv
                wvec = (ew_v[pl.ds(0, 16)], ew_v[pl.ds(16, 16)],
                        ew_v[pl.ds(32, 16)])

                acc = tuple(
                    jnp.zeros((16,), jnp.float32) for _ in range(H // 16))
                pb = nl * NP1
                for j in range(NP1):
                    wj = wvec[j // 16][j % 16]
                    wv = jnp.full((16,), wj, jnp.float32)
                    row = pb + j
                    acc = tuple(
                        acc[t] + wv * rows_v[s, row, pl.ds(t * 16, 16)]
                        for t in range(H // 16)
                    )
                for t in range(H // 16):
                    out_v[nl, pl.ds(t * 16, 16)] = acc[t]
                return carry

            lax.fori_loop(0, NB, node, 0)

            pltpu.sync_copy(out_v, out_hbm.at[pl.ds(nbase + bi * NB, NB)])

        issue(0, 0)
        issue(1, 1)

        def body(b2, carry):
            for s in range(2):
                bi = b2 * 2 + s
                compute(bi, s)
                issue(bi + 2, s)
            return carry

        lax.fori_loop(0, NBATCH // 2 - 1, body, 0)
        compute(NBATCH - 2, 0)
        compute(NBATCH - 1, 1)

    return k(emb, pq1d, idx2, idxr, mask3)


def _mlp_body(node_ref, ls_ref, fc1e_ref, fc1s_ref, fc1b_ref, fc2_ref,
              fc2b_ref, out_ref):
    h = (jnp.dot(node_ref[...], fc1e_ref[...],
                 preferred_element_type=jnp.float32)
         + jnp.dot(ls_ref[...], fc1s_ref[...],
                   preferred_element_type=jnp.float32)
         + fc1b_ref[...])
    h = jnp.where(h >= 0, h, 0.2 * h)
    o = (jnp.dot(h, fc2_ref[...], preferred_element_type=jnp.float32)
         + fc2b_ref[...])
    out_ref[...] = jnp.where(o >= 0, o, 0.2 * o)


def _tc_mlp(node, ls, fc1e, fc1s, fc1b, fc2w, fc2b):
    R = 512
    grid = (BL // R,)
    full = lambda i: (0, 0)
    return pl.pallas_call(
        _mlp_body,
        grid=grid,
        in_specs=[
            pl.BlockSpec((R, H), lambda i: (i, 0)),
            pl.BlockSpec((R, 16), lambda i: (i, 0)),
            pl.BlockSpec((H, H), full),
            pl.BlockSpec((16, H), full),
            pl.BlockSpec((1, H), full),
            pl.BlockSpec((H, H), full),
            pl.BlockSpec((1, H), full),
        ],
        out_specs=pl.BlockSpec((R, H), lambda i: (i, 0)),
        out_shape=jax.ShapeDtypeStruct((BL, H), jnp.float32),
    )(node, ls, fc1e, fc1s, fc1b, fc2w, fc2b)


def kernel(subgraph, neighs, mask, local_stats, global_stats, extra, emb,
           a_w, a_b, fc1_w, fc1_b, fc2_w, fc2_b):
    idx = jnp.concatenate(
        [subgraph.reshape(BL, 1), neighs.reshape(BL, N)], axis=1
    ).reshape(NUM_ROWS).astype(jnp.int32)
    aa = jnp.concatenate(
        [a_w[H:], a_w[:H], jnp.zeros((H, PQ - 2), jnp.float32)], axis=1)
    abcol = jnp.zeros((PQ, 1), jnp.float32).at[1, 0].set(a_b[0])
    pqT = _tc_pq(emb, aa, abcol)
    pq1d = pqT[0:2].reshape(2 * V)
    pad_a = (lax.iota(jnp.int32, BL * 15) % V).reshape(BL, 15)
    pad_b = (lax.iota(jnp.int32, BL * (SLOT - 16 - NP1)) % V).reshape(
        BL, SLOT - 16 - NP1)
    idx2 = jnp.concatenate(
        [subgraph.reshape(BL, 1).astype(jnp.int32) + V,
         pad_a,
         idx.reshape(BL, NP1),
         pad_b],
        axis=1).reshape(BL * SLOT)
    mask3 = jnp.concatenate(
        [jnp.zeros((BL, 1), jnp.float32), mask.reshape(BL, N),
         jnp.ones((BL, MW - NP1), jnp.float32)], axis=1)
    node = _sc_attention(emb, pq1d, idx2, idx, mask3)
    ls = jnp.concatenate(
        [local_stats.reshape(BL, 8),
         jnp.broadcast_to(global_stats, (B, L, 1)).reshape(BL, 1),
         jnp.zeros((BL, 7), jnp.float32)], axis=1)
    fc1e = fc1_w[:H]
    fc1s = jnp.concatenate([fc1_w[H:], jnp.zeros((7, H), jnp.float32)], axis=0)
    out = _tc_mlp(node, ls, fc1e, fc1s, fc1_b.reshape(1, H), fc2_w,
                  fc2_b.reshape(1, H))
    return out.reshape(B, L, H)


# speedup vs baseline: 6.5568x; 1.0958x over previous
import functools

import jax
import jax.numpy as jnp
from jax import lax
from jax.experimental import pallas as pl
from jax.experimental.pallas import tpu as pltpu
from jax.experimental.pallas import tpu_sc as plsc

B, L, N, H = 64, 128, 32, 128
BL = B * L
NP1 = N + 1
NUM_ROWS = BL * NP1
V = 100001
PQ = 16
NW = 32
PER_W = NUM_ROWS // NW
NODES_W = BL // NW
ACH = 4224
NB = 8
BROWS = NB * NP1
NBATCH = NODES_W // NB

def _mk_mesh():
    return plsc.VectorSubcoreMesh(
        core_axis_name="c", subcore_axis_name="s", num_cores=2, num_subcores=16
    )


def _wid():
    return lax.axis_index("s") * 2 + lax.axis_index("c")


def _pq_body(emb_ref, aa_ref, out_ref):
    out_ref[...] = lax.dot_general(
        aa_ref[...], emb_ref[...], (((0,), (1,)), ((), ())),
        preferred_element_type=jnp.float32)


def _tc_pq(emb, aa):
    R = 4096
    grid = (pl.cdiv(V, R),)
    return pl.pallas_call(
        _pq_body,
        grid=grid,
        in_specs=[
            pl.BlockSpec((R, H), lambda i: (i, 0)),
            pl.BlockSpec((H, PQ), lambda i: (0, 0)),
        ],
        out_specs=pl.BlockSpec((PQ, R), lambda i: (0, i)),
        out_shape=jax.ShapeDtypeStruct((PQ, V), jnp.float32),
    )(emb, aa)


def _sc_pq_gather(ptab, qtab, idx, qidx):
    @functools.partial(
        pl.kernel,
        out_type=(
            jax.ShapeDtypeStruct((NUM_ROWS,), jnp.float32),
            jax.ShapeDtypeStruct((BL,), jnp.float32),
        ),
        mesh=_mk_mesh(),
        scratch_types=[
            pltpu.VMEM((PER_W,), jnp.int32),
            pltpu.VMEM((PER_W,), jnp.float32),
            pltpu.VMEM((NODES_W,), jnp.int32),
            pltpu.VMEM((NODES_W,), jnp.float32),
            pltpu.SemaphoreType.DMA,
        ],
    )
    def k(ptab_hbm, qtab_hbm, idx_hbm, qidx_hbm, p_hbm, q0_hbm,
          idx_v, p_v, qidx_v, q0_v, sem):
        wid = _wid()
        base = wid * PER_W
        nb = wid * NODES_W
        pltpu.sync_copy(idx_hbm.at[pl.ds(base, PER_W)], idx_v)
        pltpu.async_copy(ptab_hbm.at[idx_v], p_v, sem).wait()
        pltpu.sync_copy(p_v, p_hbm.at[pl.ds(base, PER_W)])
        pltpu.sync_copy(qidx_hbm.at[pl.ds(nb, NODES_W)], qidx_v)
        pltpu.async_copy(qtab_hbm.at[qidx_v], q0_v, sem).wait()
        pltpu.sync_copy(q0_v, q0_hbm.at[pl.ds(nb, NODES_W)])

    return k(ptab, qtab, idx, qidx)


def _w_body(p_ref, q0_ref, mask_ref, ab_ref, out_ref):
    r = p_ref.shape[0]
    sc = q0_ref[...] + p_ref[...] + ab_ref[0, 0]
    sc = jnp.where(sc >= 0, sc, 0.2 * sc)
    neg = jnp.concatenate(
        [jnp.zeros((r, 1), jnp.float32), mask_ref[...] * (-1e9)], axis=1)
    sc = sc + neg
    m = jnp.max(sc, axis=1, keepdims=True)
    e = jnp.exp(sc - m)
    out_ref[...] = e / jnp.sum(e, axis=1, keepdims=True)


def _tc_weights(p33, q0, mask2, ab):
    R = 512
    grid = (BL // R,)
    return pl.pallas_call(
        _w_body,
        grid=grid,
        in_specs=[
            pl.BlockSpec((R, NP1), lambda i: (i, 0)),
            pl.BlockSpec((R, 1), lambda i: (i, 0)),
            pl.BlockSpec((R, N), lambda i: (i, 0)),
            pl.BlockSpec((1, 1), lambda i: (0, 0), memory_space=pltpu.SMEM),
        ],
        out_specs=pl.BlockSpec((R, NP1), lambda i: (i, 0)),
        out_shape=jax.ShapeDtypeStruct((BL, NP1), jnp.float32),
    )(p33, q0, mask2, ab)


def _sc_weighted_sum(emb, idx, w):
    @functools.partial(
        pl.kernel,
        out_type=jax.ShapeDtypeStruct((BL, H), jnp.float32),
        mesh=_mk_mesh(),
        scratch_types=[
            pltpu.VMEM((BROWS,), jnp.int32),
            pltpu.VMEM((BROWS,), jnp.int32),
            pltpu.VMEM((2, BROWS, H), jnp.float32),
            pltpu.VMEM((NODES_W, NP1), jnp.float32),
            pltpu.VMEM((NB, H), jnp.float32),
            pltpu.SemaphoreType.DMA,
            pltpu.SemaphoreType.DMA,
        ],
    )
    def k(emb_hbm, idx_hbm, w_hbm, out_hbm, idx_v0, idx_v1, rows_v, w_v,
          out_v, gsem0, gsem1):
        wid = _wid()
        rbase = wid * PER_W
        nbase = wid * NODES_W
        idxs = (idx_v0, idx_v1)
        gsems = (gsem0, gsem1)

        pltpu.sync_copy(w_hbm.at[pl.ds(nbase, NODES_W)], w_v)

        def issue(bi, s):
            pltpu.sync_copy(
                idx_hbm.at[pl.ds(rbase + bi * BROWS, BROWS)], idxs[s])
            pltpu.async_copy(emb_hbm.at[idxs[s]], rows_v.at[s], gsems[s])

        def compute(bi, s):
            pltpu.make_async_copy(
                emb_hbm.at[idxs[s]], rows_v.at[s], gsems[s]).wait()

            def node(nl, carry):
                nrow = bi * NB + nl
                wvec = (w_v[nrow, pl.ds(0, 16)],
                        w_v[nrow, pl.ds(16, 16)],
                        w_v[nrow, pl.ds(17, 16)])
                acc = tuple(
                    jnp.zeros((16,), jnp.float32) for _ in range(H // 16))
                for j in range(NP1):
                    wj = wvec[2][15] if j == 32 else wvec[j // 16][j % 16]
                    wv = jnp.full((16,), wj, dtype=jnp.float32)
                    row = nl * NP1 + j
                    acc = tuple(
                        acc[t] + wv * rows_v[s, row, pl.ds(t * 16, 16)]
                        for t in range(H // 16)
                    )
                for t in range(H // 16):
                    out_v[nl, pl.ds(t * 16, 16)] = acc[t]
                return carry

            lax.fori_loop(0, NB, node, 0)
            pltpu.sync_copy(
                out_v, out_hbm.at[pl.ds(nbase + bi * NB, NB)])

        issue(0, 0)
        issue(1, 1)

        def body(b2, carry):
            for s in range(2):
                bi = b2 * 2 + s
                compute(bi, s)
                issue(bi + 2, s)
            return carry

        lax.fori_loop(0, NBATCH // 2 - 1, body, 0)
        compute(NBATCH - 2, 0)
        compute(NBATCH - 1, 1)

    return k(emb, idx, w)


def _mlp_body(node_ref, ls_ref, fc1e_ref, fc1s_ref, fc1b_ref, fc2_ref,
              fc2b_ref, out_ref):
    h = (jnp.dot(node_ref[...], fc1e_ref[...],
                 preferred_element_type=jnp.float32)
         + jnp.dot(ls_ref[...], fc1s_ref[...],
                   preferred_element_type=jnp.float32)
         + fc1b_ref[...])
    h = jnp.where(h >= 0, h, 0.2 * h)
    o = (jnp.dot(h, fc2_ref[...], preferred_element_type=jnp.float32)
         + fc2b_ref[...])
    out_ref[...] = jnp.where(o >= 0, o, 0.2 * o)


def _tc_mlp(node, ls, fc1e, fc1s, fc1b, fc2w, fc2b):
    R = 512
    grid = (BL // R,)
    full = lambda i: (0, 0)
    return pl.pallas_call(
        _mlp_body,
        grid=grid,
        in_specs=[
            pl.BlockSpec((R, H), lambda i: (i, 0)),
            pl.BlockSpec((R, 16), lambda i: (i, 0)),
            pl.BlockSpec((H, H), full),
            pl.BlockSpec((16, H), full),
            pl.BlockSpec((1, H), full),
            pl.BlockSpec((H, H), full),
            pl.BlockSpec((1, H), full),
        ],
        out_specs=pl.BlockSpec((R, H), lambda i: (i, 0)),
        out_shape=jax.ShapeDtypeStruct((BL, H), jnp.float32),
    )(node, ls, fc1e, fc1s, fc1b, fc2w, fc2b)


def kernel(subgraph, neighs, mask, local_stats, global_stats, extra, emb,
           a_w, a_b, fc1_w, fc1_b, fc2_w, fc2_b):
    idx = jnp.concatenate(
        [subgraph.reshape(BL, 1), neighs.reshape(BL, N)], axis=1
    ).reshape(NUM_ROWS).astype(jnp.int32)
    aa = jnp.concatenate(
        [a_w[H:], a_w[:H], jnp.zeros((H, PQ - 2), jnp.float32)], axis=1)
    pqT = _tc_pq(emb, aa)
    ptab, qtab = pqT[0], pqT[1]
    p, q0 = _sc_pq_gather(ptab, qtab, idx, subgraph.reshape(BL).astype(jnp.int32))
    mask2 = mask.reshape(BL, N)
    w = _tc_weights(p.reshape(BL, NP1), q0.reshape(BL, 1), mask2,
                    a_b.reshape(1, 1))
    node = _sc_weighted_sum(emb, idx, w)
    ls = jnp.concatenate(
        [local_stats.reshape(BL, 8),
         jnp.broadcast_to(global_stats, (B, L, 1)).reshape(BL, 1),
         jnp.zeros((BL, 7), jnp.float32)], axis=1)
    fc1e = fc1_w[:H]
    fc1s = jnp.concatenate([fc1_w[H:], jnp.zeros((7, H), jnp.float32)], axis=0)
    out = _tc_mlp(node, ls, fc1e, fc1s, fc1_b.reshape(1, H), fc2_w,
                  fc2_b.reshape(1, H))
    return out.reshape(B, L, H)
